# baseline (speedup 1.0000x reference)
import jax
import jax.numpy as jnp
from jax.experimental import pallas as pl

DIM_C = 32
D = 64
SEQ = 65536
BLK = 4096


def _body(z_ref, m_ref, bias_ref, m2_ref, b2_ref, r1_ref, r2_ref, out_ref):
    zb = z_ref[...]
    lane = jax.lax.broadcasted_iota(jnp.int32, (BLK, D), 1)
    is_c = lane < DIM_C
    a = jax.lax.dot_general(zb, m_ref[...], (((1,), (1,)), ((), ())),
                            preferred_element_type=jnp.float32)
    a = a + bias_ref[...]
    e = jnp.exp(jnp.where(is_c, jnp.minimum(a, 0.0), -jnp.abs(a)))
    celu = jnp.where(a > 0, a, e - 1.0)
    sp = jnp.maximum(a, 0.0) + jnp.log(1.0 + e)
    act = jnp.where(is_c, celu, sp)
    b = jax.lax.dot_general(act, m2_ref[...], (((1,), (1,)), ((), ())),
                            preferred_element_type=jnp.float32)
    b = b + b2_ref[...]
    t = b * zb
    s = zb * zb
    nb = jnp.dot(t, r1_ref[...], preferred_element_type=jnp.float32)
    db = jnp.dot(s, r2_ref[...], preferred_element_type=jnp.float32)
    mh = jnp.where(is_c, 0.0, 1.0)
    out_ref[...] = b - (nb / db + mh * act) * zb


def kernel(t, z, F_cur_W, F_cur_b, F_out_W, F_out_b, G_W, G_b):
    m = jnp.concatenate(
        [F_cur_W, jnp.pad(G_W, ((0, 0), (0, D - DIM_C)))], axis=0)
    bias = jnp.concatenate([F_cur_b, G_b])[None, :]
    m2 = jnp.pad(F_out_W[:, :DIM_C], ((0, D - DIM_C), (0, D - DIM_C)))
    b2 = jnp.pad(F_out_b, (0, D - DIM_C))[None, :]
    lane_r = jnp.arange(D)
    r1 = ((lane_r[:, None] < DIM_C) & (lane_r[None, :] < DIM_C)).astype(jnp.float32)
    r2 = (lane_r[:, None] < DIM_C).astype(jnp.float32) * jnp.ones((1, D), jnp.float32)

    grid = (SEQ // BLK,)
    full = lambda i: (0, 0)
    out = pl.pallas_call(
        _body,
        grid=grid,
        in_specs=[
            pl.BlockSpec((BLK, None, D), lambda i: (i, 0, 0)),
            pl.BlockSpec((D, D), full),
            pl.BlockSpec((1, D), full),
            pl.BlockSpec((D, D), full),
            pl.BlockSpec((1, D), full),
            pl.BlockSpec((D, D), full),
            pl.BlockSpec((D, D), full),
        ],
        out_specs=pl.BlockSpec((BLK, None, D), lambda i: (i, 0, 0)),
        out_shape=jax.ShapeDtypeStruct((SEQ, 1, D), jnp.float32),
    )(z, m, bias, m2, b2, r1, r2)
    return out


# speedup vs baseline: 7.0705x; 7.0705x over previous
import jax
import jax.numpy as jnp
from jax.experimental import pallas as pl

DIM_C = 32
D = 64
SEQ = 65536
BLKC = 4096


def _contract(w, x):
    return jax.lax.dot_general(w, x, (((1,), (0,)), ((), ())),
                               preferred_element_type=jnp.float32)


def _body(z_ref, m_ref, bias_ref, m2_ref, b2_ref, r1_ref, r2_ref, out_ref):
    zb = z_ref[...]
    sub = jax.lax.broadcasted_iota(jnp.int32, (D, BLKC), 0)
    is_c = sub < DIM_C
    a = _contract(m_ref[...], zb) + bias_ref[...]
    e = jnp.exp(jnp.where(is_c, jnp.minimum(a, 0.0), -jnp.abs(a)))
    celu = jnp.where(a > 0, a, e - 1.0)
    sp = jnp.maximum(a, 0.0) + jnp.log(1.0 + e)
    act = jnp.where(is_c, celu, sp)
    b = _contract(m2_ref[...], act) + b2_ref[...]
    t = b * zb
    s = zb * zb
    nb = _contract(r1_ref[...], t)
    db = _contract(r2_ref[...], s)
    mh = jnp.where(is_c, 0.0, 1.0)
    out_ref[...] = b - (nb / db + mh * act) * zb


def kernel(t, z, F_cur_W, F_cur_b, F_out_W, F_out_b, G_W, G_b):
    m = jnp.concatenate(
        [F_cur_W, jnp.pad(G_W, ((0, 0), (0, D - DIM_C)))], axis=0)
    bias = jnp.concatenate([F_cur_b, G_b])[:, None]
    m2 = jnp.pad(F_out_W[:, :DIM_C], ((0, D - DIM_C), (0, D - DIM_C)))
    b2 = jnp.pad(F_out_b, (0, D - DIM_C))[:, None]
    lane_r = jnp.arange(D)
    r1 = ((lane_r[:, None] < DIM_C) & (lane_r[None, :] < DIM_C)).astype(jnp.float32)
    r2 = jnp.ones((D, 1), jnp.float32) * (lane_r[None, :] < DIM_C).astype(jnp.float32)

    zt = jnp.transpose(z, (1, 2, 0)).reshape(D, SEQ)
    grid = (SEQ // BLKC,)
    full = lambda i: (0, 0)
    out = pl.pallas_call(
        _body,
        grid=grid,
        in_specs=[
            pl.BlockSpec((D, BLKC), lambda i: (0, i)),
            pl.BlockSpec((D, D), full),
            pl.BlockSpec((D, 1), full),
            pl.BlockSpec((D, D), full),
            pl.BlockSpec((D, 1), full),
            pl.BlockSpec((D, D), full),
            pl.BlockSpec((D, D), full),
        ],
        out_specs=pl.BlockSpec((D, BLKC), lambda i: (0, i)),
        out_shape=jax.ShapeDtypeStruct((D, SEQ), jnp.float32),
    )(zt, m, bias, m2, b2, r1, r2)
    return jnp.transpose(out.reshape(1, D, SEQ), (2, 0, 1))


# speedup vs baseline: 7.9254x; 1.1209x over previous
import jax
import jax.numpy as jnp
from jax.experimental import pallas as pl

DIM_C = 32
D = 64
SEQ = 65536
BLKC = 8192


def _contract(w, x):
    return jax.lax.dot_general(w, x, (((1,), (0,)), ((), ())),
                               preferred_element_type=jnp.float32)


def _body(z_ref, m_ref, bias_ref, m2_ref, b2_ref, r1_ref, r2_ref, out_ref):
    zb = z_ref[...]
    sub = jax.lax.broadcasted_iota(jnp.int32, (D, BLKC), 0)
    is_c = sub < DIM_C
    a = _contract(m_ref[...], zb) + bias_ref[...]
    e = jnp.exp(jnp.where(is_c, jnp.minimum(a, 0.0), -jnp.abs(a)))
    celu = jnp.where(a > 0, a, e - 1.0)
    sp = jnp.maximum(a, 0.0) + jnp.log(1.0 + e)
    act = jnp.where(is_c, celu, sp)
    b = _contract(m2_ref[...], act) + b2_ref[...]
    t = b * zb
    s = zb * zb
    nb = _contract(r1_ref[...], t)
    db = _contract(r2_ref[...], s)
    mh = jnp.where(is_c, 0.0, 1.0)
    out_ref[...] = b - (nb / db + mh * act) * zb


def kernel(t, z, F_cur_W, F_cur_b, F_out_W, F_out_b, G_W, G_b):
    m = jnp.concatenate(
        [F_cur_W, jnp.pad(G_W, ((0, 0), (0, D - DIM_C)))], axis=0)
    bias = jnp.concatenate([F_cur_b, G_b])[:, None]
    m2 = jnp.pad(F_out_W[:, :DIM_C], ((0, D - DIM_C), (0, D - DIM_C)))
    b2 = jnp.pad(F_out_b, (0, D - DIM_C))[:, None]
    lane_r = jnp.arange(D)
    r1 = ((lane_r[:, None] < DIM_C) & (lane_r[None, :] < DIM_C)).astype(jnp.float32)
    r2 = jnp.ones((D, 1), jnp.float32) * (lane_r[None, :] < DIM_C).astype(jnp.float32)

    zt = jnp.transpose(z, (1, 2, 0)).reshape(D, SEQ)
    grid = (SEQ // BLKC,)
    full = lambda i: (0, 0)
    out = pl.pallas_call(
        _body,
        grid=grid,
        in_specs=[
            pl.BlockSpec((D, BLKC), lambda i: (0, i)),
            pl.BlockSpec((D, D), full),
            pl.BlockSpec((D, 1), full),
            pl.BlockSpec((D, D), full),
            pl.BlockSpec((D, 1), full),
            pl.BlockSpec((D, D), full),
            pl.BlockSpec((D, D), full),
        ],
        out_specs=pl.BlockSpec((D, BLKC), lambda i: (0, i)),
        out_shape=jax.ShapeDtypeStruct((D, SEQ), jnp.float32),
    )(zt, m, bias, m2, b2, r1, r2)
    return jnp.transpose(out.reshape(1, D, SEQ), (2, 0, 1))


# speedup vs baseline: 7.9605x; 1.0044x over previous
import jax
import jax.numpy as jnp
from jax.experimental import pallas as pl

DIM_C = 32
D = 64
SEQ = 65536
BLKC = 16384


def _contract(w, x):
    return jax.lax.dot_general(w, x, (((1,), (0,)), ((), ())),
                               preferred_element_type=jnp.float32)


def _body(z_ref, m_ref, bias_ref, m2_ref, b2_ref, r1_ref, r2_ref, out_ref):
    zb = z_ref[...]
    sub = jax.lax.broadcasted_iota(jnp.int32, (D, BLKC), 0)
    is_c = sub < DIM_C
    a = _contract(m_ref[...], zb) + bias_ref[...]
    e = jnp.exp(jnp.where(is_c, jnp.minimum(a, 0.0), -jnp.abs(a)))
    celu = jnp.where(a > 0, a, e - 1.0)
    sp = jnp.maximum(a, 0.0) + jnp.log(1.0 + e)
    act = jnp.where(is_c, celu, sp)
    b = _contract(m2_ref[...], act) + b2_ref[...]
    t = b * zb
    s = zb * zb
    nb = _contract(r1_ref[...], t)
    db = _contract(r2_ref[...], s)
    mh = jnp.where(is_c, 0.0, 1.0)
    out_ref[...] = b - (nb / db + mh * act) * zb


def kernel(t, z, F_cur_W, F_cur_b, F_out_W, F_out_b, G_W, G_b):
    m = jnp.concatenate(
        [F_cur_W, jnp.pad(G_W, ((0, 0), (0, D - DIM_C)))], axis=0)
    bias = jnp.concatenate([F_cur_b, G_b])[:, None]
    m2 = jnp.pad(F_out_W[:, :DIM_C], ((0, D - DIM_C), (0, D - DIM_C)))
    b2 = jnp.pad(F_out_b, (0, D - DIM_C))[:, None]
    lane_r = jnp.arange(D)
    r1 = ((lane_r[:, None] < DIM_C) & (lane_r[None, :] < DIM_C)).astype(jnp.float32)
    r2 = jnp.ones((D, 1), jnp.float32) * (lane_r[None, :] < DIM_C).astype(jnp.float32)

    zt = jnp.transpose(z, (1, 2, 0)).reshape(D, SEQ)
    grid = (SEQ // BLKC,)
    full = lambda i: (0, 0)
    out = pl.pallas_call(
        _body,
        grid=grid,
        in_specs=[
            pl.BlockSpec((D, BLKC), lambda i: (0, i)),
            pl.BlockSpec((D, D), full),
            pl.BlockSpec((D, 1), full),
            pl.BlockSpec((D, D), full),
            pl.BlockSpec((D, 1), full),
            pl.BlockSpec((D, D), full),
            pl.BlockSpec((D, D), full),
        ],
        out_specs=pl.BlockSpec((D, BLKC), lambda i: (0, i)),
        out_shape=jax.ShapeDtypeStruct((D, SEQ), jnp.float32),
    )(zt, m, bias, m2, b2, r1, r2)
    return jnp.transpose(out.reshape(1, D, SEQ), (2, 0, 1))


# speedup vs baseline: 11.6916x; 1.4687x over previous
import jax
import jax.numpy as jnp
from jax.experimental import pallas as pl

DIM_C = 32
D = 64
SEQ = 65536
BLKC = 8192


def _contract(w, x):
    return jax.lax.dot_general(w, x, (((1,), (0,)), ((), ())),
                               preferred_element_type=jnp.float32)


def _body(z_ref, fcw_ref, gw_ref, fow_ref, b3_ref, out_ref):
    zb = z_ref[...]
    c = zb[:DIM_C, :]
    h = zb[DIM_C:, :]
    b1 = b3_ref[:, 0:1]
    bg = b3_ref[:, 1:2]
    b2 = b3_ref[:, 2:3]
    a1 = _contract(fcw_ref[...], zb) + b1
    v1 = jnp.where(a1 > 0, a1, jnp.exp(jnp.minimum(a1, 0.0)) - 1.0)
    a2 = _contract(gw_ref[...], c) + bg
    g = jnp.maximum(a2, 0.0) + jnp.log(1.0 + jnp.exp(-jnp.abs(a2)))
    v1p = jnp.concatenate([v1, jnp.zeros_like(v1)], axis=0)
    dc = _contract(fow_ref[...], v1p) + b2
    t = dc * c
    s = c * c
    ones = jnp.ones((DIM_C, DIM_C), jnp.float32)
    nb = _contract(ones, t)
    db = _contract(ones, s)
    dcp = dc - (nb / db) * c
    out_ref[...] = jnp.concatenate([dcp, -g * h], axis=0)


def kernel(t, z, F_cur_W, F_cur_b, F_out_W, F_out_b, G_W, G_b):
    b3 = jnp.stack([F_cur_b, G_b, F_out_b], axis=1)
    zt = jnp.transpose(z, (1, 2, 0)).reshape(D, SEQ)
    grid = (SEQ // BLKC,)
    full = lambda i: (0, 0)
    out = pl.pallas_call(
        _body,
        grid=grid,
        in_specs=[
            pl.BlockSpec((D, BLKC), lambda i: (0, i)),
            pl.BlockSpec((DIM_C, D), full),
            pl.BlockSpec((DIM_C, DIM_C), full),
            pl.BlockSpec((DIM_C, D), full),
            pl.BlockSpec((DIM_C, 3), full),
        ],
        out_specs=pl.BlockSpec((D, BLKC), lambda i: (0, i)),
        out_shape=jax.ShapeDtypeStruct((D, SEQ), jnp.float32),
    )(zt, F_cur_W, G_W, F_out_W, b3)
    return jnp.transpose(out.reshape(1, D, SEQ), (2, 0, 1))
